# baseline (speedup 1.0000x reference)
import functools
import math

import jax
import jax.numpy as jnp
from jax import lax
from jax.experimental import pallas as pl
from jax.experimental.pallas import tpu as pltpu
from jax.experimental.pallas import tpu_sc as plsc

_NC = 2
_NS = 16
_LANES = 16
_NW = _NC * _NS

_CH = 128
_NBUF = 6
_K = 3


@jax.jit
def _emb_lookup(x_flat, table):
    b_total = x_flat.shape[0]
    d_model = table.shape[1]
    b_per_w = b_total // _NW
    n_chunks = b_per_w // _CH
    scale = jnp.float32(math.sqrt(float(d_model)))
    vecs_per_row = d_model // _LANES

    mesh = plsc.VectorSubcoreMesh(core_axis_name="c", subcore_axis_name="s")

    @functools.partial(
        pl.kernel,
        mesh=mesh,
        out_type=jax.ShapeDtypeStruct((b_total, d_model), jnp.float32),
        scratch_types=[
            pltpu.VMEM((b_per_w,), jnp.int32),
            [pltpu.VMEM((_CH, d_model), jnp.float32) for _ in range(_NBUF)],
            [pltpu.SemaphoreType.DMA for _ in range(_NBUF)],
            [pltpu.SemaphoreType.DMA for _ in range(_NBUF)],
        ],
    )
    def body(idx_hbm, table_hbm, out_hbm, idx_v, bufs, gsems, ssems):
        wid = lax.axis_index("s") * _NC + lax.axis_index("c")
        base = wid * b_per_w
        pltpu.sync_copy(idx_hbm.at[pl.ds(base, b_per_w)], idx_v)

        def gather(c, b):
            return pltpu.make_async_copy(
                table_hbm.at[idx_v.at[pl.ds(c * _CH, _CH)]], bufs[b], gsems[b]
            )

        def store(c, b):
            return pltpu.make_async_copy(
                bufs[b], out_hbm.at[pl.ds(base + c * _CH, _CH)], ssems[b]
            )

        def scale_buf(b):
            buf = bufs[b]

            @plsc.parallel_loop(0, _CH, unroll=2)
            def _row(r):
                for j in range(vecs_per_row):
                    sl = (r, pl.ds(j * _LANES, _LANES))
                    buf[sl] = buf[sl] * scale

        G = n_chunks
        for c in range(_K):
            gather(c, c % _NBUF).start()
        for c in range(_K):
            gather(c + _K, (c + _K) % _NBUF).start()
            gather(c, c % _NBUF).wait()
            scale_buf(c % _NBUF)
            store(c, c % _NBUF).start()
        steady = (G - 2 * _K) // _NBUF * _NBUF

        def outer(io, carry):
            for j in range(_NBUF):
                c = _K + io * _NBUF + j
                b_next = (2 * _K + j) % _NBUF
                b = (_K + j) % _NBUF
                store(c - _K, b_next).wait()
                gather(c + _K, b_next).start()
                gather(c, b).wait()
                scale_buf(b)
                store(c, b).start()
            return carry

        lax.fori_loop(0, steady // _NBUF, outer, 0)
        for c in range(_K + steady, G):
            if c + _K < G:
                store(c - _K, (c + _K) % _NBUF).wait()
                gather(c + _K, (c + _K) % _NBUF).start()
            gather(c, c % _NBUF).wait()
            scale_buf(c % _NBUF)
            store(c, c % _NBUF).start()
        for c in range(G - 2 * _K, G):
            store(c, c % _NBUF).wait()

    return body(x_flat, table)


def kernel(x, table):
    n0, n1 = x.shape
    d_model = table.shape[1]
    x_flat = x.astype(jnp.int32).T.reshape(n0 * n1)
    out = _emb_lookup(x_flat, table)
    return out.reshape(n1, n0, d_model).transpose(1, 0, 2)
